# baseline (speedup 1.0000x reference)
import jax
import jax.numpy as jnp
from jax import lax
from jax.experimental import pallas as pl
from jax.experimental.pallas import tpu as pltpu
from jax.experimental.pallas import tpu_sc as plsc

IM_SIZE = 1024
RADIUS = 128
BATCH = 512

_NW = 32
_RPW = BATCH // _NW
_NPAD = IM_SIZE + RADIUS
_NBLK = _NPAD // RADIUS
_NEG = -3.0e30
_PADV = -1000.0 - float(IM_SIZE)


def _body(hf_hbm, out_hbm, rows_v, g_v, s_v, p_v, out_v):
    wid = lax.axis_index("s") * 2 + lax.axis_index("c")
    base = wid * _RPW
    pltpu.sync_copy(hf_hbm.at[pl.ds(base, _RPW)], rows_v)

    lanes = lax.iota(jnp.int32, 16)
    negv = jnp.full((16,), _NEG, jnp.float32)

    def pass_a(jj, carries):
        j_in = 127 - jj
        new = []
        for b in range(8):
            j = b * RADIUS + j_in
            v = plsc.load_gather(rows_v, [lanes, jnp.full((16,), j, jnp.int32)])
            g = v - j.astype(jnp.float32)
            g_v[j] = g
            sm = jnp.maximum(g, carries[b])
            s_v[j] = sm
            new.append(sm)
        return tuple(new)

    lax.fori_loop(0, RADIUS, pass_a, (negv,) * 8, unroll=2)

    padv = jnp.full((16,), _PADV, jnp.float32)
    s_v[IM_SIZE] = padv

    def pass_b(j_in, carries):
        new = []
        for b in range(1, 8):
            j = b * RADIUS + j_in
            pm = jnp.maximum(g_v[j], carries[b - 1])
            p_v[j] = pm
            new.append(pm)
        p_v[IM_SIZE + j_in] = padv
        return tuple(new)

    lax.fori_loop(0, RADIUS, pass_b, (negv,) * 7, unroll=2)

    def pass_c(i, carry):
        m = jnp.maximum(s_v[i + 1], p_v[i + RADIUS])
        o = jnp.maximum(m - g_v[i], 0.0)
        plsc.store_scatter(out_v, [lanes, jnp.full((16,), i, jnp.int32)], o)
        return carry

    lax.fori_loop(0, IM_SIZE, pass_c, 0, unroll=2)

    pltpu.sync_copy(out_v, out_hbm.at[pl.ds(base, _RPW)])


def kernel(height_field):
    mesh = plsc.VectorSubcoreMesh(core_axis_name="c", subcore_axis_name="s")
    f = pl.kernel(
        _body,
        out_type=jax.ShapeDtypeStruct((BATCH, IM_SIZE), jnp.float32),
        mesh=mesh,
        scratch_types=[
            pltpu.VMEM((_RPW, IM_SIZE), jnp.float32),
            pltpu.VMEM((_NPAD, 16), jnp.float32),
            pltpu.VMEM((_NPAD, 16), jnp.float32),
            pltpu.VMEM((_NPAD, 16), jnp.float32),
            pltpu.VMEM((_RPW, IM_SIZE), jnp.float32),
        ],
        compiler_params=pltpu.CompilerParams(
            use_tc_tiling_on_sc=False, needs_layout_passes=False
        ),
    )
    return f(height_field)
